# baseline (speedup 1.0000x reference)
import jax
import jax.numpy as jnp
from jax import lax
from jax.experimental import pallas as pl
from jax.experimental.pallas import tpu as pltpu
from jax.experimental.pallas import tpu_sc as plsc

B = 4096
L = 200
DIM = 64
DPAD = 128
N = B * L
VOCAB = 1000000
SPLIT = 524288
LANES = 16

NUM_CORES = 2
NUM_SUBCORES = 16
NW = NUM_CORES * NUM_SUBCORES
PER_W = N // NW
CHUNK = 256
NCHUNK = PER_W // CHUNK
assert NCHUNK % 2 == 0

RB = 512
TGRID = SPLIT // RB


def _tpose_body(ta_ref, tb_ref, out_ref):
    out_ref[...] = jnp.concatenate([ta_ref[...].T, tb_ref[...].T], axis=1)


def _repack_table(tt):
    return pl.pallas_call(
        _tpose_body,
        grid=(TGRID,),
        in_specs=[
            pl.BlockSpec((DIM, RB), lambda i: (0, i)),
            pl.BlockSpec((DIM, RB),
                         lambda i: (0, jnp.minimum(i + TGRID,
                                                   pl.cdiv(VOCAB, RB) - 1))),
        ],
        out_specs=pl.BlockSpec((RB, DPAD), lambda i: (i, 0)),
        out_shape=jax.ShapeDtypeStruct((SPLIT, DPAD), jnp.float32),
    )(tt, tt)


def _emb_body(idx_hbm, table_hbm, out_hbm,
              idx_v0, idx_v1, rows_v0, rows_v1,
              gsem0, gsem1, osem0, osem1):
    wid = lax.axis_index("s") * NUM_CORES + lax.axis_index("c")
    base = wid * PER_W

    idx_v = (idx_v0, idx_v1)
    rows_v = (rows_v0, rows_v1)
    gsem = (gsem0, gsem1)
    osem = (osem0, osem1)

    def start_gather(g, b):
        pltpu.sync_copy(idx_hbm.at[pl.ds(base + g * CHUNK, CHUNK)], idx_v[b])
        for k in range(CHUNK // LANES):
            sl = pl.ds(k * LANES, LANES)
            r = idx_v[b][sl]
            idx_v[b][sl] = jnp.where(r < SPLIT, 2 * r,
                                     2 * r - (2 * SPLIT - 1))
        pltpu.make_async_copy(table_hbm.at[idx_v[b]], rows_v[b],
                              gsem[b]).start()

    def wait_gather(b):
        pltpu.make_async_copy(table_hbm.at[idx_v[b]], rows_v[b],
                              gsem[b]).wait()

    def start_write(g, b):
        pltpu.make_async_copy(rows_v[b],
                              out_hbm.at[pl.ds(base + g * CHUNK, CHUNK),
                                         pl.ds(0, DIM)],
                              osem[b]).start()

    def wait_write(g, b):
        pltpu.make_async_copy(rows_v[b],
                              out_hbm.at[pl.ds(base + g * CHUNK, CHUNK),
                                         pl.ds(0, DIM)],
                              osem[b]).wait()

    start_gather(0, 0)

    def body(gg, _):
        for b in range(2):
            g = gg * 2 + b
            nb = 1 - b

            @pl.when(g + 1 < NCHUNK)
            def _():
                @pl.when(g >= 1)
                def _():
                    wait_write(g - 1, nb)
                start_gather(g + 1, nb)

            wait_gather(b)
            start_write(g, b)
        return 0

    lax.fori_loop(0, NCHUNK // 2, body, 0)
    wait_write(NCHUNK - 2, 0)
    wait_write(NCHUNK - 1, 1)


def kernel(x, table):
    xf = x.reshape(N)
    tbl = _repack_table(table.T).reshape(2 * SPLIT, DIM)
    mesh = plsc.VectorSubcoreMesh(core_axis_name="c", subcore_axis_name="s",
                                  num_cores=NUM_CORES,
                                  num_subcores=NUM_SUBCORES)
    out_pad = pl.kernel(
        _emb_body,
        out_type=jax.ShapeDtypeStruct((N, DPAD), jnp.float32),
        mesh=mesh,
        scratch_types=[
            pltpu.VMEM((CHUNK,), jnp.int32),
            pltpu.VMEM((CHUNK,), jnp.int32),
            pltpu.VMEM((CHUNK, DIM), jnp.float32),
            pltpu.VMEM((CHUNK, DIM), jnp.float32),
            pltpu.SemaphoreType.DMA,
            pltpu.SemaphoreType.DMA,
            pltpu.SemaphoreType.DMA,
            pltpu.SemaphoreType.DMA,
        ],
        compiler_params=pltpu.CompilerParams(use_tc_tiling_on_sc=False),
    )(xf, tbl)
    return out_pad[:, :DIM].reshape(B, L, DIM)


# speedup vs baseline: 1.0710x; 1.0710x over previous
import jax
import jax.numpy as jnp
from jax import lax
from jax.experimental import pallas as pl
from jax.experimental.pallas import tpu as pltpu
from jax.experimental.pallas import tpu_sc as plsc

B = 4096
L = 200
DIM = 64
DPAD = 128
N = B * L
VOCAB = 1000000
SPLIT = 524288
LANES = 16

NUM_CORES = 2
NUM_SUBCORES = 16
NW = NUM_CORES * NUM_SUBCORES
PER_W = N // NW
CHUNK = 256
NCHUNK = PER_W // CHUNK
assert NCHUNK % 2 == 0

RB = 1024
TGRID = SPLIT // RB


def _tpose_body(ta_ref, tb_ref, out_ref):
    eye = jnp.eye(DIM, dtype=jnp.float32)
    dn = (((0,), (0,)), ((), ()))
    a = lax.dot_general(ta_ref[...], eye, dn,
                        preferred_element_type=jnp.float32,
                        precision=lax.Precision.HIGHEST)
    b = lax.dot_general(tb_ref[...], eye, dn,
                        preferred_element_type=jnp.float32,
                        precision=lax.Precision.HIGHEST)
    out_ref[...] = jnp.concatenate([a, b], axis=1)


def _repack_table(tt):
    return pl.pallas_call(
        _tpose_body,
        grid=(TGRID,),
        in_specs=[
            pl.BlockSpec((DIM, RB), lambda i: (0, i)),
            pl.BlockSpec((DIM, RB),
                         lambda i: (0, jnp.minimum(i + TGRID,
                                                   pl.cdiv(VOCAB, RB) - 1))),
        ],
        out_specs=pl.BlockSpec((RB, DPAD), lambda i: (i, 0)),
        out_shape=jax.ShapeDtypeStruct((SPLIT, DPAD), jnp.float32),
    )(tt, tt)


def _emb_body(idx_hbm, table_hbm, out_hbm,
              idx_v0, idx_v1, rows_v0, rows_v1,
              gsem0, gsem1, osem0, osem1):
    wid = lax.axis_index("s") * NUM_CORES + lax.axis_index("c")
    base = wid * PER_W

    idx_v = (idx_v0, idx_v1)
    rows_v = (rows_v0, rows_v1)
    gsem = (gsem0, gsem1)
    osem = (osem0, osem1)

    def start_gather(g, b):
        pltpu.sync_copy(idx_hbm.at[pl.ds(base + g * CHUNK, CHUNK)], idx_v[b])
        for k in range(CHUNK // LANES):
            sl = pl.ds(k * LANES, LANES)
            r = idx_v[b][sl]
            idx_v[b][sl] = jnp.where(r < SPLIT, 2 * r,
                                     2 * r - (2 * SPLIT - 1))
        pltpu.make_async_copy(table_hbm.at[idx_v[b]], rows_v[b],
                              gsem[b]).start()

    def wait_gather(b):
        pltpu.make_async_copy(table_hbm.at[idx_v[b]], rows_v[b],
                              gsem[b]).wait()

    def start_write(g, b):
        pltpu.make_async_copy(rows_v[b],
                              out_hbm.at[pl.ds(base + g * CHUNK, CHUNK),
                                         pl.ds(0, DIM)],
                              osem[b]).start()

    def wait_write(g, b):
        pltpu.make_async_copy(rows_v[b],
                              out_hbm.at[pl.ds(base + g * CHUNK, CHUNK),
                                         pl.ds(0, DIM)],
                              osem[b]).wait()

    start_gather(0, 0)

    def body(gg, _):
        for b in range(2):
            g = gg * 2 + b
            nb = 1 - b

            @pl.when(g + 1 < NCHUNK)
            def _():
                @pl.when(g >= 1)
                def _():
                    wait_write(g - 1, nb)
                start_gather(g + 1, nb)

            wait_gather(b)
            start_write(g, b)
        return 0

    lax.fori_loop(0, NCHUNK // 2, body, 0)
    wait_write(NCHUNK - 2, 0)
    wait_write(NCHUNK - 1, 1)


def kernel(x, table):
    xf = x.reshape(N)
    tbl = _repack_table(table.T).reshape(2 * SPLIT, DIM)
    mesh = plsc.VectorSubcoreMesh(core_axis_name="c", subcore_axis_name="s",
                                  num_cores=NUM_CORES,
                                  num_subcores=NUM_SUBCORES)
    out_pad = pl.kernel(
        _emb_body,
        out_type=jax.ShapeDtypeStruct((N, DPAD), jnp.float32),
        mesh=mesh,
        scratch_types=[
            pltpu.VMEM((CHUNK,), jnp.int32),
            pltpu.VMEM((CHUNK,), jnp.int32),
            pltpu.VMEM((CHUNK, DIM), jnp.float32),
            pltpu.VMEM((CHUNK, DIM), jnp.float32),
            pltpu.SemaphoreType.DMA,
            pltpu.SemaphoreType.DMA,
            pltpu.SemaphoreType.DMA,
            pltpu.SemaphoreType.DMA,
        ],
        compiler_params=pltpu.CompilerParams(use_tc_tiling_on_sc=False),
    )(xf, tbl)
    return out_pad[:, :DIM].reshape(B, L, DIM)


# speedup vs baseline: 1.6865x; 1.5748x over previous
import jax
import jax.numpy as jnp
from jax import lax
from jax.experimental import pallas as pl
from jax.experimental.pallas import tpu as pltpu
from jax.experimental.pallas import tpu_sc as plsc

B = 4096
L = 200
DIM = 64
DPAD = 128
N = B * L
VOCAB = 1000000
SPLIT = 524288
LANES = 16

NUM_CORES = 2
NUM_SUBCORES = 16
NW = NUM_CORES * NUM_SUBCORES
PER_W = N // NW
CHUNK = 256
NCHUNK = PER_W // CHUNK
assert NCHUNK % 2 == 0

RB = 2048
TGRID = SPLIT // RB


def _tpose_body(ta_ref, tb_ref, out_ref):
    m = jnp.concatenate([ta_ref[...], tb_ref[...]], axis=0)
    out_ref[...] = m.T


def _repack_table(tt):
    return pl.pallas_call(
        _tpose_body,
        grid=(TGRID,),
        in_specs=[
            pl.BlockSpec((DIM, RB), lambda i: (0, i)),
            pl.BlockSpec((DIM, RB),
                         lambda i: (0, jnp.minimum(i + TGRID,
                                                   pl.cdiv(VOCAB, RB) - 1))),
        ],
        out_specs=pl.BlockSpec((RB, DPAD), lambda i: (i, 0)),
        out_shape=jax.ShapeDtypeStruct((SPLIT, DPAD), jnp.float32),
    )(tt, tt)


def _emb_body(idx_hbm, table_hbm, out_hbm,
              idx_v0, idx_v1, rows_v0, rows_v1,
              gsem0, gsem1, osem0, osem1):
    wid = lax.axis_index("s") * NUM_CORES + lax.axis_index("c")
    base = wid * PER_W

    idx_v = (idx_v0, idx_v1)
    rows_v = (rows_v0, rows_v1)
    gsem = (gsem0, gsem1)
    osem = (osem0, osem1)

    def start_gather(g, b):
        pltpu.sync_copy(idx_hbm.at[pl.ds(base + g * CHUNK, CHUNK)], idx_v[b])
        for k in range(CHUNK // LANES):
            sl = pl.ds(k * LANES, LANES)
            r = idx_v[b][sl]
            idx_v[b][sl] = jnp.where(r < SPLIT, 2 * r,
                                     2 * r - (2 * SPLIT - 1))
        pltpu.make_async_copy(table_hbm.at[idx_v[b]], rows_v[b],
                              gsem[b]).start()

    def wait_gather(b):
        pltpu.make_async_copy(table_hbm.at[idx_v[b]], rows_v[b],
                              gsem[b]).wait()

    def start_write(g, b):
        pltpu.make_async_copy(rows_v[b],
                              out_hbm.at[pl.ds(base + g * CHUNK, CHUNK),
                                         pl.ds(0, DIM)],
                              osem[b]).start()

    def wait_write(g, b):
        pltpu.make_async_copy(rows_v[b],
                              out_hbm.at[pl.ds(base + g * CHUNK, CHUNK),
                                         pl.ds(0, DIM)],
                              osem[b]).wait()

    start_gather(0, 0)

    def body(gg, _):
        for b in range(2):
            g = gg * 2 + b
            nb = 1 - b

            @pl.when(g + 1 < NCHUNK)
            def _():
                @pl.when(g >= 1)
                def _():
                    wait_write(g - 1, nb)
                start_gather(g + 1, nb)

            wait_gather(b)
            start_write(g, b)
        return 0

    lax.fori_loop(0, NCHUNK // 2, body, 0)
    wait_write(NCHUNK - 2, 0)
    wait_write(NCHUNK - 1, 1)


def kernel(x, table):
    xf = x.reshape(N)
    tbl = _repack_table(table.T).reshape(2 * SPLIT, DIM)
    mesh = plsc.VectorSubcoreMesh(core_axis_name="c", subcore_axis_name="s",
                                  num_cores=NUM_CORES,
                                  num_subcores=NUM_SUBCORES)
    out_pad = pl.kernel(
        _emb_body,
        out_type=jax.ShapeDtypeStruct((N, DPAD), jnp.float32),
        mesh=mesh,
        scratch_types=[
            pltpu.VMEM((CHUNK,), jnp.int32),
            pltpu.VMEM((CHUNK,), jnp.int32),
            pltpu.VMEM((CHUNK, DIM), jnp.float32),
            pltpu.VMEM((CHUNK, DIM), jnp.float32),
            pltpu.SemaphoreType.DMA,
            pltpu.SemaphoreType.DMA,
            pltpu.SemaphoreType.DMA,
            pltpu.SemaphoreType.DMA,
        ],
        compiler_params=pltpu.CompilerParams(use_tc_tiling_on_sc=False),
    )(xf, tbl)
    return out_pad[:, :DIM].reshape(B, L, DIM)


# speedup vs baseline: 1.9654x; 1.1653x over previous
import jax
import jax.numpy as jnp
from jax import lax
from jax.experimental import pallas as pl
from jax.experimental.pallas import tpu as pltpu
from jax.experimental.pallas import tpu_sc as plsc

B = 4096
L = 200
DIM = 64
DPAD = 128
N = B * L
VOCAB = 1000000
SPLIT = 524288
LANES = 16

NUM_CORES = 2
NUM_SUBCORES = 16
NW = NUM_CORES * NUM_SUBCORES
PER_W = N // NW
CHUNK = 512
NCHUNK = PER_W // CHUNK
assert NCHUNK % 2 == 0

RB = 4096
TGRID = SPLIT // RB


def _tpose_body(ta_ref, tb_ref, out_ref):
    m = jnp.concatenate([ta_ref[...], tb_ref[...]], axis=0)
    out_ref[...] = m.T


def _repack_table(tt):
    return pl.pallas_call(
        _tpose_body,
        grid=(TGRID,),
        in_specs=[
            pl.BlockSpec((DIM, RB), lambda i: (0, i)),
            pl.BlockSpec((DIM, RB),
                         lambda i: (0, jnp.minimum(i + TGRID,
                                                   pl.cdiv(VOCAB, RB) - 1))),
        ],
        out_specs=pl.BlockSpec((RB, DPAD), lambda i: (i, 0)),
        out_shape=jax.ShapeDtypeStruct((SPLIT, DPAD), jnp.float32),
    )(tt, tt)


def _emb_body(idx_hbm, table_hbm, out_hbm,
              idx_v0, idx_v1, rows_v0, rows_v1,
              gsem0, gsem1, osem0, osem1):
    wid = lax.axis_index("s") * NUM_CORES + lax.axis_index("c")
    base = wid * PER_W

    idx_v = (idx_v0, idx_v1)
    rows_v = (rows_v0, rows_v1)
    gsem = (gsem0, gsem1)
    osem = (osem0, osem1)

    def start_gather(g, b):
        pltpu.sync_copy(idx_hbm.at[pl.ds(base + g * CHUNK, CHUNK)], idx_v[b])
        for k in range(CHUNK // LANES):
            sl = pl.ds(k * LANES, LANES)
            r = idx_v[b][sl]
            idx_v[b][sl] = jnp.where(r < SPLIT, 2 * r,
                                     2 * r - (2 * SPLIT - 1))
        pltpu.make_async_copy(table_hbm.at[idx_v[b]], rows_v[b],
                              gsem[b]).start()

    def wait_gather(b):
        pltpu.make_async_copy(table_hbm.at[idx_v[b]], rows_v[b],
                              gsem[b]).wait()

    def start_write(g, b):
        pltpu.make_async_copy(rows_v[b],
                              out_hbm.at[pl.ds(base + g * CHUNK, CHUNK),
                                         pl.ds(0, DIM)],
                              osem[b]).start()

    def wait_write(g, b):
        pltpu.make_async_copy(rows_v[b],
                              out_hbm.at[pl.ds(base + g * CHUNK, CHUNK),
                                         pl.ds(0, DIM)],
                              osem[b]).wait()

    start_gather(0, 0)

    def body(gg, _):
        for b in range(2):
            g = gg * 2 + b
            nb = 1 - b

            @pl.when(g + 1 < NCHUNK)
            def _():
                @pl.when(g >= 1)
                def _():
                    wait_write(g - 1, nb)
                start_gather(g + 1, nb)

            wait_gather(b)
            start_write(g, b)
        return 0

    lax.fori_loop(0, NCHUNK // 2, body, 0)
    wait_write(NCHUNK - 2, 0)
    wait_write(NCHUNK - 1, 1)


def kernel(x, table):
    xf = x.reshape(N)
    tbl = _repack_table(table.T).reshape(2 * SPLIT, DIM)
    mesh = plsc.VectorSubcoreMesh(core_axis_name="c", subcore_axis_name="s",
                                  num_cores=NUM_CORES,
                                  num_subcores=NUM_SUBCORES)
    out_pad = pl.kernel(
        _emb_body,
        out_type=jax.ShapeDtypeStruct((N, DPAD), jnp.float32),
        mesh=mesh,
        scratch_types=[
            pltpu.VMEM((CHUNK,), jnp.int32),
            pltpu.VMEM((CHUNK,), jnp.int32),
            pltpu.VMEM((CHUNK, DIM), jnp.float32),
            pltpu.VMEM((CHUNK, DIM), jnp.float32),
            pltpu.SemaphoreType.DMA,
            pltpu.SemaphoreType.DMA,
            pltpu.SemaphoreType.DMA,
            pltpu.SemaphoreType.DMA,
        ],
        compiler_params=pltpu.CompilerParams(use_tc_tiling_on_sc=False),
    )(xf, tbl)
    return out_pad[:, :DIM].reshape(B, L, DIM)


# speedup vs baseline: 2.0736x; 1.0551x over previous
import jax
import jax.numpy as jnp
from jax import lax
from jax.experimental import pallas as pl
from jax.experimental.pallas import tpu as pltpu
from jax.experimental.pallas import tpu_sc as plsc

B = 4096
L = 200
DIM = 64
DPAD = 128
N = B * L
VOCAB = 1000000
SPLIT = 524288
LANES = 16

NUM_CORES = 2
NUM_SUBCORES = 16
NW = NUM_CORES * NUM_SUBCORES
PER_W = N // NW
CHUNK = 512
NCHUNK = PER_W // CHUNK
assert NCHUNK % 2 == 0

RB = 8192
TGRID = SPLIT // RB


def _tpose_body(ta_ref, tb_ref, out_ref):
    m = jnp.concatenate([ta_ref[...], tb_ref[...]], axis=0)
    out_ref[...] = m.T


def _repack_table(tt):
    return pl.pallas_call(
        _tpose_body,
        grid=(TGRID,),
        in_specs=[
            pl.BlockSpec((DIM, RB), lambda i: (0, i)),
            pl.BlockSpec((DIM, RB),
                         lambda i: (0, jnp.minimum(i + TGRID,
                                                   pl.cdiv(VOCAB, RB) - 1))),
        ],
        out_specs=pl.BlockSpec((RB, DPAD), lambda i: (i, 0)),
        out_shape=jax.ShapeDtypeStruct((SPLIT, DPAD), jnp.float32),
    )(tt, tt)


def _emb_body(idx_hbm, table_hbm, out_hbm,
              idx_v0, idx_v1, rows_v0, rows_v1,
              gsem0, gsem1, osem0, osem1):
    wid = lax.axis_index("s") * NUM_CORES + lax.axis_index("c")
    base = wid * PER_W

    idx_v = (idx_v0, idx_v1)
    rows_v = (rows_v0, rows_v1)
    gsem = (gsem0, gsem1)
    osem = (osem0, osem1)

    def start_gather(g, b):
        pltpu.sync_copy(idx_hbm.at[pl.ds(base + g * CHUNK, CHUNK)], idx_v[b])
        for k in range(CHUNK // LANES):
            sl = pl.ds(k * LANES, LANES)
            r = idx_v[b][sl]
            idx_v[b][sl] = jnp.where(r < SPLIT, 2 * r,
                                     2 * r - (2 * SPLIT - 1))
        pltpu.make_async_copy(table_hbm.at[idx_v[b]], rows_v[b],
                              gsem[b]).start()

    def wait_gather(b):
        pltpu.make_async_copy(table_hbm.at[idx_v[b]], rows_v[b],
                              gsem[b]).wait()

    def start_write(g, b):
        pltpu.make_async_copy(rows_v[b],
                              out_hbm.at[pl.ds(base + g * CHUNK, CHUNK),
                                         pl.ds(0, DIM)],
                              osem[b]).start()

    def wait_write(g, b):
        pltpu.make_async_copy(rows_v[b],
                              out_hbm.at[pl.ds(base + g * CHUNK, CHUNK),
                                         pl.ds(0, DIM)],
                              osem[b]).wait()

    start_gather(0, 0)

    def body(gg, _):
        for b in range(2):
            g = gg * 2 + b
            nb = 1 - b

            @pl.when(g + 1 < NCHUNK)
            def _():
                @pl.when(g >= 1)
                def _():
                    wait_write(g - 1, nb)
                start_gather(g + 1, nb)

            wait_gather(b)
            start_write(g, b)
        return 0

    lax.fori_loop(0, NCHUNK // 2, body, 0)
    wait_write(NCHUNK - 2, 0)
    wait_write(NCHUNK - 1, 1)


def kernel(x, table):
    xf = x.reshape(N)
    tbl = _repack_table(table.T).reshape(2 * SPLIT, DIM)
    mesh = plsc.VectorSubcoreMesh(core_axis_name="c", subcore_axis_name="s",
                                  num_cores=NUM_CORES,
                                  num_subcores=NUM_SUBCORES)
    out_pad = pl.kernel(
        _emb_body,
        out_type=jax.ShapeDtypeStruct((N, DPAD), jnp.float32),
        mesh=mesh,
        scratch_types=[
            pltpu.VMEM((CHUNK,), jnp.int32),
            pltpu.VMEM((CHUNK,), jnp.int32),
            pltpu.VMEM((CHUNK, DIM), jnp.float32),
            pltpu.VMEM((CHUNK, DIM), jnp.float32),
            pltpu.SemaphoreType.DMA,
            pltpu.SemaphoreType.DMA,
            pltpu.SemaphoreType.DMA,
            pltpu.SemaphoreType.DMA,
        ],
        compiler_params=pltpu.CompilerParams(use_tc_tiling_on_sc=False),
    )(xf, tbl)
    return out_pad[:, :DIM].reshape(B, L, DIM)


# speedup vs baseline: 2.0999x; 1.0126x over previous
import jax
import jax.numpy as jnp
from jax import lax
from jax.experimental import pallas as pl
from jax.experimental.pallas import tpu as pltpu
from jax.experimental.pallas import tpu_sc as plsc

B = 4096
L = 200
DIM = 64
DPAD = 128
N = B * L
VOCAB = 1000000
SPLIT = 524288
LANES = 16

NUM_CORES = 2
NUM_SUBCORES = 16
NW = NUM_CORES * NUM_SUBCORES
PER_W = N // NW
CHUNK = 512
NCHUNK = PER_W // CHUNK
assert NCHUNK % 2 == 0

RB = 16384
TGRID = SPLIT // RB


def _tpose_body(ta_ref, tb_ref, out_ref):
    m = jnp.concatenate([ta_ref[...], tb_ref[...]], axis=0)
    out_ref[...] = m.T


def _repack_table(tt):
    return pl.pallas_call(
        _tpose_body,
        grid=(TGRID,),
        in_specs=[
            pl.BlockSpec((DIM, RB), lambda i: (0, i)),
            pl.BlockSpec((DIM, RB),
                         lambda i: (0, jnp.minimum(i + TGRID,
                                                   pl.cdiv(VOCAB, RB) - 1))),
        ],
        out_specs=pl.BlockSpec((RB, DPAD), lambda i: (i, 0)),
        out_shape=jax.ShapeDtypeStruct((SPLIT, DPAD), jnp.float32),
    )(tt, tt)


def _emb_body(idx_hbm, table_hbm, out_hbm,
              idx_v0, idx_v1, rows_v0, rows_v1,
              gsem0, gsem1, osem0, osem1):
    wid = lax.axis_index("s") * NUM_CORES + lax.axis_index("c")
    base = wid * PER_W

    idx_v = (idx_v0, idx_v1)
    rows_v = (rows_v0, rows_v1)
    gsem = (gsem0, gsem1)
    osem = (osem0, osem1)

    def start_gather(g, b):
        pltpu.sync_copy(idx_hbm.at[pl.ds(base + g * CHUNK, CHUNK)], idx_v[b])
        for k in range(CHUNK // LANES):
            sl = pl.ds(k * LANES, LANES)
            r = idx_v[b][sl]
            idx_v[b][sl] = jnp.where(r < SPLIT, 2 * r,
                                     2 * r - (2 * SPLIT - 1))
        pltpu.make_async_copy(table_hbm.at[idx_v[b]], rows_v[b],
                              gsem[b]).start()

    def wait_gather(b):
        pltpu.make_async_copy(table_hbm.at[idx_v[b]], rows_v[b],
                              gsem[b]).wait()

    def start_write(g, b):
        pltpu.make_async_copy(rows_v[b],
                              out_hbm.at[pl.ds(base + g * CHUNK, CHUNK),
                                         pl.ds(0, DIM)],
                              osem[b]).start()

    def wait_write(g, b):
        pltpu.make_async_copy(rows_v[b],
                              out_hbm.at[pl.ds(base + g * CHUNK, CHUNK),
                                         pl.ds(0, DIM)],
                              osem[b]).wait()

    start_gather(0, 0)

    def body(gg, _):
        for b in range(2):
            g = gg * 2 + b
            nb = 1 - b

            @pl.when(g + 1 < NCHUNK)
            def _():
                @pl.when(g >= 1)
                def _():
                    wait_write(g - 1, nb)
                start_gather(g + 1, nb)

            wait_gather(b)
            start_write(g, b)
        return 0

    lax.fori_loop(0, NCHUNK // 2, body, 0)
    wait_write(NCHUNK - 2, 0)
    wait_write(NCHUNK - 1, 1)


def kernel(x, table):
    xf = x.reshape(N)
    tbl = _repack_table(table.T).reshape(2 * SPLIT, DIM)
    mesh = plsc.VectorSubcoreMesh(core_axis_name="c", subcore_axis_name="s",
                                  num_cores=NUM_CORES,
                                  num_subcores=NUM_SUBCORES)
    out_pad = pl.kernel(
        _emb_body,
        out_type=jax.ShapeDtypeStruct((N, DPAD), jnp.float32),
        mesh=mesh,
        scratch_types=[
            pltpu.VMEM((CHUNK,), jnp.int32),
            pltpu.VMEM((CHUNK,), jnp.int32),
            pltpu.VMEM((CHUNK, DIM), jnp.float32),
            pltpu.VMEM((CHUNK, DIM), jnp.float32),
            pltpu.SemaphoreType.DMA,
            pltpu.SemaphoreType.DMA,
            pltpu.SemaphoreType.DMA,
            pltpu.SemaphoreType.DMA,
        ],
        compiler_params=pltpu.CompilerParams(use_tc_tiling_on_sc=False),
    )(xf, tbl)
    return out_pad[:, :DIM].reshape(B, L, DIM)
